# Initial kernel scaffold; baseline (speedup 1.0000x reference)
import jax
import jax.numpy as jnp
from jax.experimental import pallas as pl


def kernel(z, embedding_table):
    raise NotImplementedError("write your pallas kernel here")


# speedup vs baseline: 1.9182x; 1.9182x over previous
import jax
import jax.numpy as jnp
from jax.experimental import pallas as pl
from jax.experimental.pallas import tpu as pltpu

_B, _C, _H, _W = 8, 32, 64, 64
_S = _H * _W
_K = 1024
_TOK = 512
_NT = _S // _TOK


def _vq_body(z_ref, emb_ref, embt_ref, zq_ref, ids_ref, loss_ref):
    zb = z_ref[0]
    emb = emb_ref[...]
    embt = embt_ref[...]

    scores = jax.lax.dot_general(
        emb, zb, (((1,), (0,)), ((), ())),
        preferred_element_type=jnp.float32)
    e2 = jnp.sum(emb * emb, axis=1, keepdims=True)
    z2 = jnp.sum(zb * zb, axis=0, keepdims=True)
    d = (z2 + e2) - 2.0 * scores

    dmin = jnp.min(d, axis=0)
    rows = jax.lax.broadcasted_iota(jnp.int32, (_K, _TOK), 0)
    ids = jnp.min(jnp.where(d == dmin[None, :], rows, _K), axis=0)

    oh = (rows == ids[None, :]).astype(jnp.float32)
    zq = jax.lax.dot_general(
        embt, oh, (((1,), (0,)), ((), ())),
        preferred_element_type=jnp.float32)

    zq_ref[0] = zq
    ids_ref[0, 0] = ids

    diff = zq - zb
    part = jnp.sum(diff * diff)

    @pl.when((pl.program_id(0) == 0) & (pl.program_id(1) == 0))
    def _init():
        loss_ref[0, 0] = 0.0

    loss_ref[0, 0] += part


def kernel(z, embedding_table):
    z = z.astype(jnp.float32)
    z3 = z.reshape(_B, _C, _S)
    emb = embedding_table.astype(jnp.float32)
    embt = emb.T

    zq3, ids2, loss_sum = pl.pallas_call(
        _vq_body,
        grid=(_B, _NT),
        in_specs=[
            pl.BlockSpec((1, _C, _TOK), lambda b, t: (b, 0, t)),
            pl.BlockSpec((_K, _C), lambda b, t: (0, 0)),
            pl.BlockSpec((_C, _K), lambda b, t: (0, 0)),
        ],
        out_specs=[
            pl.BlockSpec((1, _C, _TOK), lambda b, t: (b, 0, t)),
            pl.BlockSpec((1, 1, _TOK), lambda b, t: (b * _NT + t, 0, 0)),
            pl.BlockSpec(memory_space=pltpu.SMEM),
        ],
        out_shape=[
            jax.ShapeDtypeStruct((_B, _C, _S), jnp.float32),
            jax.ShapeDtypeStruct((_B * _NT, 1, _TOK), jnp.int32),
            jax.ShapeDtypeStruct((1, 1), jnp.float32),
        ],
    )(z3, emb, embt)

    zq = zq3.reshape(_B, _C, _H, _W)
    ids = ids2.reshape(_B * _S)
    mse = loss_sum[0, 0] / (_B * _C * _S)
    commitment_loss = 0.25 * mse
    codebook_loss = mse
    loss = commitment_loss + codebook_loss
    return (zq, loss, commitment_loss, codebook_loss, ids)


# speedup vs baseline: 2.1446x; 1.1181x over previous
import jax
import jax.numpy as jnp
from jax.experimental import pallas as pl
from jax.experimental.pallas import tpu as pltpu

_B, _C, _H, _W = 8, 32, 64, 64
_S = _H * _W
_K = 1024
_TOK = 512
_NT = _S // _TOK


def _vq_body(z_ref, emb_ref, embt_ref, zq_ref, ids_ref, loss_ref):
    zb = z_ref[0]
    emb = emb_ref[...]
    embt = embt_ref[...]

    scores_m2 = jax.lax.dot_general(
        emb * (-2.0), zb, (((1,), (0,)), ((), ())),
        preferred_element_type=jnp.float32)
    e2 = jnp.sum(emb * emb, axis=1, keepdims=True)
    z2 = jnp.sum(zb * zb, axis=0, keepdims=True)
    d = (z2 + e2) + scores_m2

    dmin = jnp.min(d, axis=0)
    rowsf = jax.lax.broadcasted_iota(jnp.int32, (_K, _TOK), 0).astype(jnp.float32)
    w = jnp.where(d == dmin[None, :], rowsf, float(2 * _K))
    idsf = jnp.min(w, axis=0)

    oh = jnp.where(w == idsf[None, :], 1.0, 0.0)
    zq = jax.lax.dot_general(
        embt, oh, (((1,), (0,)), ((), ())),
        preferred_element_type=jnp.float32)

    zq_ref[0] = zq
    ids_ref[0, 0] = idsf.astype(jnp.int32)

    part = jnp.sum(dmin)

    @pl.when((pl.program_id(0) == 0) & (pl.program_id(1) == 0))
    def _init():
        loss_ref[0, 0] = 0.0

    loss_ref[0, 0] += part


def kernel(z, embedding_table):
    z = z.astype(jnp.float32)
    z3 = z.reshape(_B, _C, _S)
    emb = embedding_table.astype(jnp.float32)
    embt = emb.T

    zq3, ids2, loss_sum = pl.pallas_call(
        _vq_body,
        grid=(_B, _NT),
        in_specs=[
            pl.BlockSpec((1, _C, _TOK), lambda b, t: (b, 0, t)),
            pl.BlockSpec((_K, _C), lambda b, t: (0, 0)),
            pl.BlockSpec((_C, _K), lambda b, t: (0, 0)),
        ],
        out_specs=[
            pl.BlockSpec((1, _C, _TOK), lambda b, t: (b, 0, t)),
            pl.BlockSpec((1, 1, _TOK), lambda b, t: (b * _NT + t, 0, 0)),
            pl.BlockSpec(memory_space=pltpu.SMEM),
        ],
        out_shape=[
            jax.ShapeDtypeStruct((_B, _C, _S), jnp.float32),
            jax.ShapeDtypeStruct((_B * _NT, 1, _TOK), jnp.int32),
            jax.ShapeDtypeStruct((1, 1), jnp.float32),
        ],
    )(z3, emb, embt)

    zq = zq3.reshape(_B, _C, _H, _W)
    ids = ids2.reshape(_B * _S)
    mse = loss_sum[0, 0] / (_B * _C * _S)
    commitment_loss = 0.25 * mse
    codebook_loss = mse
    loss = commitment_loss + codebook_loss
    return (zq, loss, commitment_loss, codebook_loss, ids)


# speedup vs baseline: 2.7437x; 1.2794x over previous
import jax
import jax.numpy as jnp
from jax.experimental import pallas as pl
from jax.experimental.pallas import tpu as pltpu

_B, _C, _H, _W = 8, 32, 64, 64
_S = _H * _W
_K = 1024
_TOK = 4096
_NT = _S // _TOK


def _vq_body(z_ref, emb_ref, embt_ref, zq_ref, ids_ref, loss_ref):
    zb = z_ref[0]
    emb = emb_ref[...]
    embt = embt_ref[...]

    scores_m2 = jax.lax.dot_general(
        emb * (-2.0), zb, (((1,), (0,)), ((), ())),
        preferred_element_type=jnp.float32)
    e2 = jnp.sum(emb * emb, axis=1, keepdims=True)
    z2 = jnp.sum(zb * zb, axis=0, keepdims=True)
    d = (z2 + e2) + scores_m2

    dmin = jnp.min(d, axis=0)
    rowsf = jax.lax.broadcasted_iota(jnp.int32, (_K, _TOK), 0).astype(jnp.float32)
    w = jnp.where(d == dmin[None, :], rowsf, float(2 * _K))
    idsf = jnp.min(w, axis=0)

    oh = jnp.where(w == idsf[None, :], 1.0, 0.0)
    zq = jax.lax.dot_general(
        embt, oh, (((1,), (0,)), ((), ())),
        preferred_element_type=jnp.float32)

    zq_ref[0] = zq
    ids_ref[0, 0] = idsf.astype(jnp.int32)

    part = jnp.sum(dmin)

    @pl.when((pl.program_id(0) == 0) & (pl.program_id(1) == 0))
    def _init():
        loss_ref[0, 0] = 0.0

    loss_ref[0, 0] += part


def kernel(z, embedding_table):
    z = z.astype(jnp.float32)
    z3 = z.reshape(_B, _C, _S)
    emb = embedding_table.astype(jnp.float32)
    embt = emb.T

    zq3, ids2, loss_sum = pl.pallas_call(
        _vq_body,
        grid=(_B, _NT),
        in_specs=[
            pl.BlockSpec((1, _C, _TOK), lambda b, t: (b, 0, t)),
            pl.BlockSpec((_K, _C), lambda b, t: (0, 0)),
            pl.BlockSpec((_C, _K), lambda b, t: (0, 0)),
        ],
        out_specs=[
            pl.BlockSpec((1, _C, _TOK), lambda b, t: (b, 0, t)),
            pl.BlockSpec((1, 1, _TOK), lambda b, t: (b * _NT + t, 0, 0)),
            pl.BlockSpec(memory_space=pltpu.SMEM),
        ],
        out_shape=[
            jax.ShapeDtypeStruct((_B, _C, _S), jnp.float32),
            jax.ShapeDtypeStruct((_B * _NT, 1, _TOK), jnp.int32),
            jax.ShapeDtypeStruct((1, 1), jnp.float32),
        ],
    )(z3, emb, embt)

    zq = zq3.reshape(_B, _C, _H, _W)
    ids = ids2.reshape(_B * _S)
    mse = loss_sum[0, 0] / (_B * _C * _S)
    commitment_loss = 0.25 * mse
    codebook_loss = mse
    loss = commitment_loss + codebook_loss
    return (zq, loss, commitment_loss, codebook_loss, ids)
